# initial kernel scaffold (unmeasured)
import jax
import jax.numpy as jnp
from jax import lax
from jax.experimental import pallas as pl
from jax.experimental.pallas import tpu as pltpu


def kernel(
    x,
):
    def body(*refs):
        pass

    out_shape = jax.ShapeDtypeStruct(..., jnp.float32)
    return pl.pallas_call(body, out_shape=out_shape)(...)


# baseline (device time: 407668 ns/iter reference)
import jax
import jax.numpy as jnp
from jax import lax
from jax.experimental import pallas as pl
from jax.experimental.pallas import tpu as pltpu


def kernel(x):
    m, n = x.shape
    half = m // 2

    def body(x_ref, out_ref, comm_ref, send1, recv1, send2, recv2, cp_sem):
        my_x = lax.axis_index("x")
        my_y = lax.axis_index("y")
        my_z = lax.axis_index("z")
        partner_x = (1 - my_x, my_y, my_z)
        partner_z = (my_x, my_y, 1 - my_z)

        zlo = my_z * half

        barrier_sem = pltpu.get_barrier_semaphore()
        for nbr in (partner_x, partner_z):
            pl.semaphore_signal(
                barrier_sem, inc=1,
                device_id=nbr, device_id_type=pl.DeviceIdType.MESH,
            )
        pl.semaphore_wait(barrier_sem, 2)

        rdma1 = pltpu.make_async_remote_copy(
            src_ref=x_ref.at[pl.ds(zlo, half), :],
            dst_ref=comm_ref,
            send_sem=send1,
            recv_sem=recv1,
            device_id=partner_x,
            device_id_type=pl.DeviceIdType.MESH,
        )
        rdma1.start()
        rdma1.wait()

        comm_ref[...] = comm_ref[...] + x_ref[pl.ds(zlo, half), :]

        cp = pltpu.make_async_copy(
            comm_ref, out_ref.at[pl.ds(zlo, half), :], cp_sem
        )
        cp.start()
        cp.wait()

        rdma2 = pltpu.make_async_remote_copy(
            src_ref=comm_ref,
            dst_ref=out_ref.at[pl.ds(zlo, half), :],
            send_sem=send2,
            recv_sem=recv2,
            device_id=partner_z,
            device_id_type=pl.DeviceIdType.MESH,
        )
        rdma2.start()
        rdma2.wait()

    return pl.pallas_call(
        body,
        out_shape=jax.ShapeDtypeStruct((m, n), x.dtype),
        in_specs=[pl.BlockSpec(memory_space=pltpu.VMEM)],
        out_specs=pl.BlockSpec(memory_space=pl.ANY),
        scratch_shapes=[
            pltpu.VMEM((half, n), x.dtype),
            pltpu.SemaphoreType.DMA,
            pltpu.SemaphoreType.DMA,
            pltpu.SemaphoreType.DMA,
            pltpu.SemaphoreType.DMA,
            pltpu.SemaphoreType.DMA,
        ],
        compiler_params=pltpu.CompilerParams(
            collective_id=0,
            vmem_limit_bytes=56 * 1024 * 1024,
        ),
    )(x)


# device time: 232202 ns/iter; 1.7557x vs baseline; 1.7557x over previous
import jax
import jax.numpy as jnp
from jax import lax
from jax.experimental import pallas as pl
from jax.experimental.pallas import tpu as pltpu

N_CHUNKS = 16


def kernel(x):
    m, n = x.shape
    half = m // 2
    rows = half // N_CHUNKS

    def body(x_ref, out_ref, comm_ref, send1, recv1, send2, recv2, cp_sems):
        my_x = lax.axis_index("x")
        my_y = lax.axis_index("y")
        my_z = lax.axis_index("z")
        partner_x = (1 - my_x, my_y, my_z)
        partner_z = (my_x, my_y, 1 - my_z)

        zlo = my_z * half

        barrier_sem = pltpu.get_barrier_semaphore()
        for nbr in (partner_x, partner_z):
            pl.semaphore_signal(
                barrier_sem, inc=1,
                device_id=nbr, device_id_type=pl.DeviceIdType.MESH,
            )
        pl.semaphore_wait(barrier_sem, 2)

        rdma1 = []
        for c in range(N_CHUNKS):
            r = pltpu.make_async_remote_copy(
                src_ref=x_ref.at[pl.ds(zlo + c * rows, rows), :],
                dst_ref=comm_ref.at[pl.ds(c * rows, rows), :],
                send_sem=send1.at[c],
                recv_sem=recv1.at[c],
                device_id=partner_x,
                device_id_type=pl.DeviceIdType.MESH,
            )
            r.start()
            rdma1.append(r)

        rdma2 = []
        cps = []
        for c in range(N_CHUNKS):
            rdma1[c].wait_recv()
            csl = pl.ds(c * rows, rows)
            comm_ref[csl, :] = comm_ref[csl, :] + x_ref[pl.ds(zlo + c * rows, rows), :]
            r2 = pltpu.make_async_remote_copy(
                src_ref=comm_ref.at[csl, :],
                dst_ref=out_ref.at[pl.ds(zlo + c * rows, rows), :],
                send_sem=send2.at[c],
                recv_sem=recv2.at[c],
                device_id=partner_z,
                device_id_type=pl.DeviceIdType.MESH,
            )
            r2.start()
            rdma2.append(r2)
            cp = pltpu.make_async_copy(
                comm_ref.at[csl, :],
                out_ref.at[pl.ds(zlo + c * rows, rows), :],
                cp_sems.at[c],
            )
            cp.start()
            cps.append(cp)

        for c in range(N_CHUNKS):
            rdma1[c].wait_send()
            rdma2[c].wait()
            cps[c].wait()

    return pl.pallas_call(
        body,
        out_shape=jax.ShapeDtypeStruct((m, n), x.dtype),
        in_specs=[pl.BlockSpec(memory_space=pltpu.VMEM)],
        out_specs=pl.BlockSpec(memory_space=pl.ANY),
        scratch_shapes=[
            pltpu.VMEM((half, n), x.dtype),
            pltpu.SemaphoreType.DMA((N_CHUNKS,)),
            pltpu.SemaphoreType.DMA((N_CHUNKS,)),
            pltpu.SemaphoreType.DMA((N_CHUNKS,)),
            pltpu.SemaphoreType.DMA((N_CHUNKS,)),
            pltpu.SemaphoreType.DMA((N_CHUNKS,)),
        ],
        compiler_params=pltpu.CompilerParams(
            collective_id=0,
            vmem_limit_bytes=56 * 1024 * 1024,
        ),
    )(x)


# device time: 188666 ns/iter; 2.1608x vs baseline; 1.2308x over previous
import jax
import jax.numpy as jnp
from jax import lax
from jax.experimental import pallas as pl
from jax.experimental.pallas import tpu as pltpu

QROWS = 2048
CH = 256
NCK = QROWS // CH
NDIAG_X = 2
FWD_Y = (2, 3, 4)
FWD_Z = (5, 6, 7)


def kernel(x):
    m, n = x.shape

    def body(
        x_ref, out_ref, bown_ref, bdiag_ref,
        sx_send, sx_recv,
        s2_send, s2_recv,
        s3_send, s3_recv,
        s4_send, s4_recv,
        s5_send, s5_recv,
        cp_sems,
    ):
        my_x = lax.axis_index("x")
        my_y = lax.axis_index("y")
        my_z = lax.axis_index("z")
        xn = (1 - my_x, my_y, my_z)
        yn = (my_x, 1 - my_y, my_z)
        zn = (my_x, my_y, 1 - my_z)

        q = 2 * my_y + my_z
        yq = 2 * (1 - my_y) + my_z
        zq = 2 * my_y + (1 - my_z)
        dq = 3 - q

        def rows(quarter, k):
            return pl.ds(quarter * QROWS + k * CH, CH)

        barrier_sem = pltpu.get_barrier_semaphore()
        for nbr in (xn, yn, zn):
            pl.semaphore_signal(
                barrier_sem, inc=1,
                device_id=nbr, device_id_type=pl.DeviceIdType.MESH,
            )
        pl.semaphore_wait(barrier_sem, 3)

        f1 = []
        for k in range(NCK):
            r = pltpu.make_async_remote_copy(
                src_ref=x_ref.at[rows(q, k), :],
                dst_ref=bown_ref.at[pl.ds(k * CH, CH), :],
                send_sem=sx_send.at[k],
                recv_sem=sx_recv.at[k],
                device_id=xn,
                device_id_type=pl.DeviceIdType.MESH,
            )
            r.start()
            f1.append(r)
        for j in range(NDIAG_X):
            r = pltpu.make_async_remote_copy(
                src_ref=x_ref.at[rows(dq, j), :],
                dst_ref=bdiag_ref.at[pl.ds(j * CH, CH), :],
                send_sem=sx_send.at[NCK + j],
                recv_sem=sx_recv.at[NCK + j],
                device_id=xn,
                device_id_type=pl.DeviceIdType.MESH,
            )
            r.start()
            f1.append(r)

        f2, f3, cps = [], [], []
        for k in range(NCK):
            f1[k].wait_recv()
            ksl = pl.ds(k * CH, CH)
            bown_ref[ksl, :] = bown_ref[ksl, :] + x_ref[rows(q, k), :]
            for sems, lst, nbr in (
                ((s2_send, s2_recv), f2, yn),
                ((s3_send, s3_recv), f3, zn),
            ):
                r = pltpu.make_async_remote_copy(
                    src_ref=bown_ref.at[ksl, :],
                    dst_ref=out_ref.at[rows(q, k), :],
                    send_sem=sems[0].at[k],
                    recv_sem=sems[1].at[k],
                    device_id=nbr,
                    device_id_type=pl.DeviceIdType.MESH,
                )
                r.start()
                lst.append(r)
            cp = pltpu.make_async_copy(
                bown_ref.at[ksl, :], out_ref.at[rows(q, k), :], cp_sems.at[k]
            )
            cp.start()
            cps.append(cp)

        for j in range(NDIAG_X):
            f1[NCK + j].wait_recv()
            jsl = pl.ds(j * CH, CH)
            bdiag_ref[jsl, :] = bdiag_ref[jsl, :] + x_ref[rows(dq, j), :]
            cp = pltpu.make_async_copy(
                bdiag_ref.at[jsl, :], out_ref.at[rows(dq, j), :],
                cp_sems.at[NCK + j],
            )
            cp.start()
            cps.append(cp)

        def recv_mirror(quarter, k, recv_sem, send_sem):
            return pltpu.make_async_remote_copy(
                src_ref=bown_ref.at[pl.ds(k * CH, CH), :],
                dst_ref=out_ref.at[rows(quarter, k), :],
                send_sem=send_sem,
                recv_sem=recv_sem,
                device_id=xn,
                device_id_type=pl.DeviceIdType.MESH,
            )

        m2 = [recv_mirror(yq, k, s2_recv.at[k], s2_send.at[k]) for k in range(NCK)]
        m3 = [recv_mirror(zq, k, s3_recv.at[k], s3_send.at[k]) for k in range(NCK)]

        f4 = []
        for i, f in enumerate(FWD_Y):
            m3[f].wait_recv()
            r = pltpu.make_async_remote_copy(
                src_ref=out_ref.at[rows(zq, f), :],
                dst_ref=out_ref.at[rows(zq, f), :],
                send_sem=s4_send.at[i],
                recv_sem=s4_recv.at[i],
                device_id=yn,
                device_id_type=pl.DeviceIdType.MESH,
            )
            r.start()
            f4.append(r)

        f5 = []
        for i, f in enumerate(FWD_Z):
            m2[f].wait_recv()
            r = pltpu.make_async_remote_copy(
                src_ref=out_ref.at[rows(yq, f), :],
                dst_ref=out_ref.at[rows(yq, f), :],
                send_sem=s5_send.at[i],
                recv_sem=s5_recv.at[i],
                device_id=zn,
                device_id_type=pl.DeviceIdType.MESH,
            )
            r.start()
            f5.append(r)

        for k in range(NCK):
            if k not in FWD_Z:
                m2[k].wait_recv()
            if k not in FWD_Y:
                m3[k].wait_recv()
        for i, f in enumerate(FWD_Y):
            recv_mirror(dq, f, s4_recv.at[i], s4_send.at[i]).wait_recv()
        for i, f in enumerate(FWD_Z):
            recv_mirror(dq, f, s5_recv.at[i], s5_send.at[i]).wait_recv()
        for r in f1 + f2 + f3 + f4 + f5:
            r.wait_send()
        for cp in cps:
            cp.wait()

    return pl.pallas_call(
        body,
        out_shape=jax.ShapeDtypeStruct((m, n), x.dtype),
        in_specs=[pl.BlockSpec(memory_space=pltpu.VMEM)],
        out_specs=pl.BlockSpec(memory_space=pl.ANY),
        scratch_shapes=[
            pltpu.VMEM((QROWS, n), x.dtype),
            pltpu.VMEM((NDIAG_X * CH, n), x.dtype),
            pltpu.SemaphoreType.DMA((NCK + NDIAG_X,)),
            pltpu.SemaphoreType.DMA((NCK + NDIAG_X,)),
            pltpu.SemaphoreType.DMA((NCK,)),
            pltpu.SemaphoreType.DMA((NCK,)),
            pltpu.SemaphoreType.DMA((NCK,)),
            pltpu.SemaphoreType.DMA((NCK,)),
            pltpu.SemaphoreType.DMA((len(FWD_Y),)),
            pltpu.SemaphoreType.DMA((len(FWD_Y),)),
            pltpu.SemaphoreType.DMA((len(FWD_Z),)),
            pltpu.SemaphoreType.DMA((len(FWD_Z),)),
            pltpu.SemaphoreType.DMA((NCK + NDIAG_X,)),
        ],
        compiler_params=pltpu.CompilerParams(
            collective_id=0,
            vmem_limit_bytes=56 * 1024 * 1024,
        ),
    )(x)
